# Initial kernel scaffold; baseline (speedup 1.0000x reference)
import jax
import jax.numpy as jnp
from jax.experimental import pallas as pl


def kernel(x, edge_index, pos_edge_index, neg_edge_index, W_self1, W_neigh1, b1, W_self2, W_neigh2, b2, W_self3, W_neigh3, b3, P1_W, P1_b, P2_W, P2_b, P3_W, P3_b):
    raise NotImplementedError("write your pallas kernel here")


# speedup vs baseline: 5.6940x; 5.6940x over previous
import functools

import jax
import jax.numpy as jnp
from jax import lax
from jax.experimental import pallas as pl
from jax.experimental.pallas import tpu as pltpu
from jax.experimental.pallas import tpu_sc as plsc

N = 10000
NP = 10240
E = 320000
D = 128
HALF = 64
P = 16384

N_CORES = 2
N_SUB = 16
ROWS_PER_SUB = NP // N_SUB

EDGES_PER_SUB = E // N_SUB
AGG_CHUNK = 400
AGG_CHUNKS = EDGES_PER_SUB // AGG_CHUNK

DEG_W = 16
EDGES_PER_TILE = E // (N_CORES * N_SUB)
DEG_CHUNK = 1000
DEG_CHUNKS = EDGES_PER_TILE // DEG_CHUNK

GIDX = 4 * P
GROWS_PER_TILE = GIDX // (N_CORES * N_SUB)
G_CHUNK = 512
G_CHUNKS = GROWS_PER_TILE // G_CHUNK

_MESH = plsc.VectorSubcoreMesh(core_axis_name="c", subcore_axis_name="s")


def _zero_rows(ref, nrows, width):
    zv = jnp.zeros((16,), jnp.float32)

    def body(i, _):
        for j in range(width // 16):
            ref[i, pl.ds(j * 16, 16)] = zv
        return 0

    lax.fori_loop(0, nrows, body, 0)


@functools.partial(
    pl.kernel,
    out_type=jax.ShapeDtypeStruct((N_CORES, NP, HALF), jnp.float32),
    mesh=_MESH,
    scratch_types=[
        pltpu.VMEM_SHARED((NP, HALF), jnp.float32),
        pltpu.VMEM_SHARED((NP, HALF), jnp.float32),
        pltpu.VMEM((AGG_CHUNK,), jnp.int32),
        pltpu.VMEM((AGG_CHUNK,), jnp.int32),
        pltpu.VMEM((AGG_CHUNK, HALF), jnp.float32),
        pltpu.SemaphoreType.DMA,
    ],
    compiler_params=pltpu.CompilerParams(use_tc_tiling_on_sc=False),
)
def _sc_aggregate(hT, src, dst, out, table, accum, srcv, dstv, msgs, sem):
    c = lax.axis_index("c")
    s = lax.axis_index("s")
    r0 = s * ROWS_PER_SUB

    pltpu.sync_copy(hT.at[c, pl.ds(r0, ROWS_PER_SUB)],
                    table.at[pl.ds(r0, ROWS_PER_SUB)])
    _zero_rows(msgs, AGG_CHUNK, HALF)
    off = 0
    while off < ROWS_PER_SUB:
        nb = min(AGG_CHUNK, ROWS_PER_SUB - off)
        pltpu.sync_copy(msgs.at[pl.ds(0, nb)],
                        accum.at[pl.ds(r0 + off, nb)])
        off += nb
    plsc.subcore_barrier()

    base = s * EDGES_PER_SUB

    def chunk(k, _):
        off = base + k * AGG_CHUNK
        pltpu.sync_copy(src.at[pl.ds(off, AGG_CHUNK)], srcv)
        pltpu.sync_copy(dst.at[pl.ds(off, AGG_CHUNK)], dstv)
        pltpu.async_copy(table.at[srcv], msgs, sem).wait()
        pltpu.sync_copy(msgs, accum.at[dstv], add=True)
        return 0

    lax.fori_loop(0, AGG_CHUNKS, chunk, 0)
    plsc.subcore_barrier()
    pltpu.sync_copy(accum.at[pl.ds(r0, ROWS_PER_SUB)],
                    out.at[c, pl.ds(r0, ROWS_PER_SUB)])


@functools.partial(
    pl.kernel,
    out_type=jax.ShapeDtypeStruct((N_CORES, NP, DEG_W), jnp.float32),
    mesh=_MESH,
    scratch_types=[
        pltpu.VMEM_SHARED((NP, DEG_W), jnp.float32),
        pltpu.VMEM((DEG_CHUNK, DEG_W), jnp.float32),
        pltpu.VMEM((DEG_CHUNK,), jnp.int32),
    ],
    compiler_params=pltpu.CompilerParams(use_tc_tiling_on_sc=False),
)
def _sc_degree(dst, out, accum, buf, dstv):
    c = lax.axis_index("c")
    s = lax.axis_index("s")
    r0 = s * ROWS_PER_SUB

    _zero_rows(buf, DEG_CHUNK, DEG_W)
    pltpu.sync_copy(buf.at[pl.ds(0, ROWS_PER_SUB)],
                    accum.at[pl.ds(r0, ROWS_PER_SUB)])

    ov = jnp.ones((16,), jnp.float32)

    def fill(i, _):
        buf[i, :] = ov
        return 0

    lax.fori_loop(0, DEG_CHUNK, fill, 0)
    plsc.subcore_barrier()

    base = (s * N_CORES + c) * EDGES_PER_TILE

    def chunk(k, _):
        pltpu.sync_copy(dst.at[pl.ds(base + k * DEG_CHUNK, DEG_CHUNK)], dstv)
        pltpu.sync_copy(buf, accum.at[dstv], add=True)
        return 0

    lax.fori_loop(0, DEG_CHUNKS, chunk, 0)
    plsc.subcore_barrier()
    pltpu.sync_copy(accum.at[pl.ds(r0, ROWS_PER_SUB)],
                    out.at[c, pl.ds(r0, ROWS_PER_SUB)])


@functools.partial(
    pl.kernel,
    out_type=jax.ShapeDtypeStruct((GIDX, D), jnp.float32),
    mesh=_MESH,
    scratch_types=[
        pltpu.VMEM((GROWS_PER_TILE,), jnp.int32),
        pltpu.VMEM((G_CHUNK, D), jnp.float32),
        pltpu.SemaphoreType.DMA,
    ],
    compiler_params=pltpu.CompilerParams(use_tc_tiling_on_sc=False),
)
def _sc_gather(h, idx, out, idxv, rows, sem):
    c = lax.axis_index("c")
    s = lax.axis_index("s")
    base = (s * N_CORES + c) * GROWS_PER_TILE
    pltpu.sync_copy(idx.at[pl.ds(base, GROWS_PER_TILE)], idxv)
    for k in range(G_CHUNKS):
        pltpu.async_copy(h.at[idxv.at[pl.ds(k * G_CHUNK, G_CHUNK)]],
                         rows, sem).wait()
        pltpu.sync_copy(rows, out.at[pl.ds(base + k * G_CHUNK, G_CHUNK)])


def _dense_body(relu, h_ref, agg_ref, deg_ref, ws_ref, wn_ref, b_ref, o_ref):
    deg = deg_ref[0, :, 0:1] + deg_ref[1, :, 0:1]
    inv = 1.0 / jnp.maximum(deg, 1.0)
    y = (jnp.dot(h_ref[...], ws_ref[...], preferred_element_type=jnp.float32)
         + jnp.dot(agg_ref[...], wn_ref[...],
                   preferred_element_type=jnp.float32) * inv
         + b_ref[...])
    o_ref[...] = jnp.maximum(y, 0.0) if relu else y


def _tc_dense(h, agg, deg2, ws, wn, b, relu):
    bm = 1000
    return pl.pallas_call(
        functools.partial(_dense_body, relu),
        grid=(N // bm,),
        in_specs=[
            pl.BlockSpec((bm, D), lambda i: (i, 0)),
            pl.BlockSpec((bm, D), lambda i: (i, 0)),
            pl.BlockSpec((N_CORES, bm, DEG_W), lambda i: (0, i, 0)),
            pl.BlockSpec((D, D), lambda i: (0, 0)),
            pl.BlockSpec((D, D), lambda i: (0, 0)),
            pl.BlockSpec((1, D), lambda i: (0, 0)),
        ],
        out_specs=pl.BlockSpec((bm, D), lambda i: (i, 0)),
        out_shape=jax.ShapeDtypeStruct((N, D), jnp.float32),
    )(h, agg, deg2, ws, wn, b.reshape(1, D))


def _mlp_body(a_ref, b_ref, w1, b1, w2, b2, w3, b3, o_ref):
    t = a_ref[...] * b_ref[...]
    t = jnp.maximum(jnp.dot(t, w1[...], preferred_element_type=jnp.float32)
                    + b1[...], 0.0)
    t = jnp.maximum(jnp.dot(t, w2[...], preferred_element_type=jnp.float32)
                    + b2[...], 0.0)
    o_ref[...] = (jnp.dot(t, w3[...], preferred_element_type=jnp.float32)
                  + b3[...])


def _tc_mlp(a, b, p1w, p1b, p2w, p2b, p3w_pad, p3b_pad):
    bm = 1024
    wspec = pl.BlockSpec((D, D), lambda i: (0, 0))
    bspec = pl.BlockSpec((1, D), lambda i: (0, 0))
    return pl.pallas_call(
        _mlp_body,
        grid=(P // bm,),
        in_specs=[pl.BlockSpec((bm, D), lambda i: (i, 0)),
                  pl.BlockSpec((bm, D), lambda i: (i, 0)),
                  wspec, bspec, wspec, bspec, wspec, bspec],
        out_specs=pl.BlockSpec((bm, D), lambda i: (i, 0)),
        out_shape=jax.ShapeDtypeStruct((P, D), jnp.float32),
    )(a, b, p1w, p1b.reshape(1, D), p2w, p2b.reshape(1, D), p3w_pad, p3b_pad)


def kernel(x, edge_index, pos_edge_index, neg_edge_index,
           W_self1, W_neigh1, b1, W_self2, W_neigh2, b2,
           W_self3, W_neigh3, b3,
           P1_W, P1_b, P2_W, P2_b, P3_W, P3_b):
    src = edge_index[0].astype(jnp.int32)
    dst = edge_index[1].astype(jnp.int32)

    deg2 = _sc_degree(dst)[:, :N, :]

    h = x
    for ws, wn, b, relu in ((W_self1, W_neigh1, b1, True),
                            (W_self2, W_neigh2, b2, True),
                            (W_self3, W_neigh3, b3, False)):
        hp = jnp.pad(h, ((0, NP - N), (0, 0)))
        hT = jnp.transpose(hp.reshape(NP, N_CORES, HALF), (1, 0, 2))
        aggT = _sc_aggregate(hT, src, dst)
        agg = jnp.transpose(aggT, (1, 0, 2)).reshape(NP, D)[:N]
        h = _tc_dense(h, agg, deg2, ws, wn, b, relu)

    idx4 = jnp.concatenate([pos_edge_index[0], pos_edge_index[1],
                            neg_edge_index[0], neg_edge_index[1]]).astype(jnp.int32)
    g = _sc_gather(h, idx4)

    p3w_pad = jnp.pad(P3_W, ((0, 0), (0, D - 1)))
    p3b_pad = jnp.pad(P3_b, (0, D - 1)).reshape(1, D)
    out_pos = _tc_mlp(g[0:P], g[P:2 * P], P1_W, P1_b, P2_W, P2_b,
                      p3w_pad, p3b_pad)
    out_neg = _tc_mlp(g[2 * P:3 * P], g[3 * P:4 * P], P1_W, P1_b,
                      P2_W, P2_b, p3w_pad, p3b_pad)
    return (out_pos[:, 0:1], out_neg[:, 0:1])
